# baseline (speedup 1.0000x reference)
import dataclasses

import jax
import jax.numpy as jnp
from jax import lax
from jax.experimental import pallas as pl
from jax.experimental.pallas import tpu as pltpu
from jax.experimental.pallas import tpu_sc as plsc
from jax._src.pallas import mpmd
from jax._src.pallas import core as pallas_core
from jax._src.pallas.mosaic import core as tpu_core

_NUM_ACT = 192
_SIZE = 256
_RANGE = 4.0
_GRID = 2.0 * _RANGE / (_SIZE - 1)
_INV_GRID = 1.0 / _GRID
_HALF_GRID = _GRID / 2.0

_H = 224
_SLAB = _H * _H
_N = 2 * _NUM_ACT * _SLAB
_TBL = _NUM_ACT * _SIZE

_NC, _NS = 2, 16
_CHUNK = _SLAB // 4
_CHUNKS_PER_SLAB = _SLAB // _CHUNK
_GROUP = _NS * _CHUNK
_ROUNDS = _N // (_NC * _GROUP)


def _prep_body(cs_ref, tbl_ref):
    cs = cs_ref[...]
    cs_next = jnp.concatenate([cs[:, 1:], cs[:, _SIZE - 1:]], axis=1)
    slopes = jnp.maximum(cs_next - cs, 0.0)
    col = lax.broadcasted_iota(jnp.int32, (_NUM_ACT, _SIZE), 1)
    slopes = jnp.where((col == 0) | (col >= _SIZE - 2), 0.0, slopes)
    r = lax.broadcasted_iota(jnp.int32, (_SIZE, _SIZE), 0)
    c = lax.broadcasted_iota(jnp.int32, (_SIZE, _SIZE), 1)
    tri = (r < c).astype(jnp.float32)
    new_cs = jnp.dot(slopes, tri, preferred_element_type=jnp.float32)
    centered = new_cs - new_cs[:, _SIZE // 2:_SIZE // 2 + 1]
    a2 = centered + _HALF_GRID
    a2n = jnp.concatenate([a2[:, 1:], a2[:, _SIZE - 1:]], axis=1)
    d = a2n - a2
    ah = lax.bitcast_convert_type(
        a2.astype(jnp.bfloat16).astype(jnp.float32), jnp.int32)
    dh = lax.bitcast_convert_type(
        d.astype(jnp.bfloat16).astype(jnp.float32), jnp.int32)
    tbl_ref[...] = ah | lax.shift_right_logical(dh, 16)


def _scs_body(x_hbm, tbl_hbm, out_hbm, spin, spout,
              in_rdy0, in_rdy1, out_fre0, out_fre1,
              in_fre0, in_fre1, out_rdy0, out_rdy1):
    cid = lax.axis_index("c")
    in_rdy, out_fre = (in_rdy0, in_rdy1), (out_fre0, out_fre1)
    in_fre, out_rdy = (in_fre0, in_fre1), (out_rdy0, out_rdy1)
    base = cid * _ROUNDS * _GROUP

    def bcast(sem, inc):
        for t in range(_NS):
            pltpu.semaphore_signal(sem, inc, device_id={"s": t})

    def gsrc(r):
        return x_hbm.at[pl.ds(base + r * _GROUP, _GROUP)]

    def gdst(r):
        return out_hbm.at[pl.ds(base + r * _GROUP, _GROUP)]

    bcast(out_fre0, 1)
    bcast(out_fre1, 1)
    pltpu.sync_copy(gsrc(0), spin.at[0])
    bcast(in_rdy0, 1)
    pltpu.sync_copy(gsrc(1), spin.at[1])
    bcast(in_rdy1, 1)

    def pair_body(pr, carry):
        for b in (0, 1):
            r = pr * 2 + b

            @pl.when(r + 2 < _ROUNDS)
            def _():
                pltpu.semaphore_wait(in_fre[b], _NS)
                pltpu.sync_copy(gsrc(r + 2), spin.at[b])
                bcast(in_rdy[b], 1)

            pltpu.semaphore_wait(out_rdy[b], _NS)
            pltpu.sync_copy(spout.at[b], gdst(r))
            bcast(out_fre[b], 1)
        return carry

    lax.fori_loop(0, _ROUNDS // 2, pair_body, jnp.int32(0))
    pltpu.semaphore_wait(in_fre0, _NS)
    pltpu.semaphore_wait(in_fre1, _NS)


def _tec_body(x_hbm, tbl_hbm, out_hbm, spin, spout,
              in_rdy0, in_rdy1, out_fre0, out_fre1,
              in_fre0, in_fre1, out_rdy0, out_rdy1):
    pl.run_scoped(
        lambda tbl_v, iv: _tec_inner(
            x_hbm, tbl_hbm, out_hbm, tbl_v, iv, spin, spout,
            in_rdy0, in_rdy1, out_fre0, out_fre1,
            in_fre0, in_fre1, out_rdy0, out_rdy1),
        pltpu.VMEM((_TBL,), jnp.int32),
        pltpu.VMEM((_CHUNK,), jnp.float32),
    )


def _tec_inner(x_hbm, tbl_hbm, out_hbm, tbl_v, iv, spin, spout,
               in_rdy0, in_rdy1, out_fre0, out_fre1,
               in_fre0, in_fre1, out_rdy0, out_rdy1):
    cid = lax.axis_index("c")
    sid = lax.axis_index("s")
    in_rdy, out_fre = (in_rdy0, in_rdy1), (out_fre0, out_fre1)
    in_fre, out_rdy = (in_fre0, in_fre1), (out_rdy0, out_rdy1)
    pltpu.sync_copy(tbl_hbm, tbl_v)
    g0 = (cid * _ROUNDS) * _NS + sid

    def pair_body(pr, carry):
        for b in (0, 1):
            r = pr * 2 + b
            g = g0 + r * _NS
            ch = (g // _CHUNKS_PER_SLAB) % _NUM_ACT
            base_v = jnp.full((16,), ch * _SIZE, jnp.int32)
            pltpu.semaphore_wait(in_rdy[b], 1)
            pltpu.sync_copy(spin.at[b, pl.ds(sid * _CHUNK, _CHUNK)], iv)
            pltpu.semaphore_signal(in_fre[b], 1)

            def vbody(i):
                xv = iv[pl.ds(i, 16)]
                u_raw = xv * _INV_GRID + (_SIZE / 2 - 0.5)
                u_cl = jnp.minimum(jnp.maximum(u_raw, 0.0), float(_SIZE - 3))
                iu = u_cl.astype(jnp.int32)
                fr = u_raw - iu.astype(jnp.float32)
                idx = iu + base_v
                w = plsc.load_gather(tbl_v, [idx])
                a2f = plsc.bitcast(w & jnp.int32(-65536), jnp.float32)
                df = plsc.bitcast(lax.shift_left(w, 16), jnp.float32)
                iv[pl.ds(i, 16)] = a2f + fr * df

            plsc.parallel_loop(0, _CHUNK, step=16, unroll=8)(vbody)
            pltpu.semaphore_wait(out_fre[b], 1)
            pltpu.sync_copy(iv, spout.at[b, pl.ds(sid * _CHUNK, _CHUNK)])
            pltpu.semaphore_signal(out_rdy[b], 1)
        return carry

    lax.fori_loop(0, _ROUNDS // 2, pair_body, jnp.int32(0))
    pltpu.semaphore_wait(out_fre0, 1)
    pltpu.semaphore_wait(out_fre1, 1)


def _make_sc():
    smesh = plsc.ScalarSubcoreMesh(axis_name="c", num_cores=_NC)
    vmesh = plsc.VectorSubcoreMesh(
        core_axis_name="c", subcore_axis_name="s",
        num_cores=_NC, num_subcores=_NS)

    def sem(mesh_):
        ref = tpu_core.SemaphoreType.REGULAR(())
        return dataclasses.replace(
            ref, memory_space=pallas_core.CoreMemorySpace(
                tpu_core.MemorySpace.SEMAPHORE, mesh_))

    return mpmd.mpmd_map(
        [(smesh, _scs_body), (vmesh, _tec_body)],
        out_types=jax.ShapeDtypeStruct((_N,), jnp.float32),
        scratch_types=[
            tpu_core.MemorySpace.VMEM_SHARED((2, _GROUP), jnp.float32),
            tpu_core.MemorySpace.VMEM_SHARED((2, _GROUP), jnp.float32),
            sem(vmesh), sem(vmesh), sem(vmesh), sem(vmesh),
            sem(smesh), sem(smesh), sem(smesh), sem(smesh),
        ],
        compiler_params=pltpu.CompilerParams(needs_layout_passes=False),
    )


def kernel(x, coefficients_vect):
    cs = coefficients_vect.reshape(_NUM_ACT, _SIZE)
    tbl = pl.pallas_call(
        _prep_body,
        out_shape=jax.ShapeDtypeStruct((_NUM_ACT, _SIZE), jnp.int32),
    )(cs)
    out_flat = _make_sc()(x.reshape(_N), tbl.reshape(_TBL))
    return out_flat.reshape(x.shape)
